# baseline (speedup 1.0000x reference)
import jax
import jax.numpy as jnp
from jax.experimental import pallas as pl
from jax.experimental.pallas import tpu as pltpu

N = 4096
D_IN = 256
D_HID = 128
K_OPP = 11
B = 256
GK = N // B


def _layer1_stripe(k, a_ref, e_ref, W1_ref, W2_ref, b1_ref,
                   up_ref, dinv_ref, z_s, mp_s, upb_s, store_bf):
    a = a_ref[...]
    ab = a.astype(jnp.bfloat16)
    colr = jnp.sum(a, axis=0, keepdims=True)
    dinv_c = jnp.transpose(1.0 / (colr + 1.0))
    dinv_ref[pl.ds(k * B, B), :] = dinv_c
    m = jnp.dot(e_ref[...].astype(jnp.bfloat16), W1_ref[...].astype(jnp.bfloat16),
                preferred_element_type=jnp.float32)
    mp = dinv_c * m
    mp_s[pl.ds(k * B, B), :] = mp
    zp = jnp.dot(ab, mp.astype(jnp.bfloat16), preferred_element_type=jnp.float32)

    @pl.when(k == 0)
    def _():
        z_s[...] = zp

    @pl.when(k != 0)
    def _():
        z_s[...] += zp

    @pl.when(k == GK - 1)
    def _():
        dinv = dinv_ref[...]
        xm = jax.nn.sigmoid(dinv * (z_s[...] + mp_s[...]) + b1_ref[...])
        u = jnp.dot(xm, W2_ref[...], preferred_element_type=jnp.float32)
        up = dinv * u
        up_ref[...] = up
        if store_bf:
            upb_s[...] = up.astype(jnp.bfloat16)


def _gcn_body(A1c_ref, A2c_ref, A2r_ref, E1_ref, E2_ref, W1_ref, W2_ref,
              b1_ref, b2_ref,
              upx_ref, dinvx_ref, upy_ref, dinvy_ref, G_ref,
              z_s, mp_s, upb_s):
    p = pl.program_id(0)
    k = pl.program_id(1)

    @pl.when(p == 0)
    def _():
        _layer1_stripe(k, A1c_ref, E1_ref, W1_ref, W2_ref, b1_ref,
                       upx_ref, dinvx_ref, z_s, mp_s, upb_s, False)

    @pl.when(p == 1)
    def _():
        _layer1_stripe(k, A2c_ref, E2_ref, W1_ref, W2_ref, b1_ref,
                       upy_ref, dinvy_ref, z_s, mp_s, upb_s, True)

    @pl.when(p == 2)
    def _():
        ab = A2r_ref[...].astype(jnp.bfloat16)
        w = jnp.dot(ab, upb_s[...], preferred_element_type=jnp.float32)
        dinv_b = dinvy_ref[pl.ds(k * B, B), :]
        up_b = upy_ref[pl.ds(k * B, B), :]
        G_ref[pl.ds(k * B, B), :] = jax.nn.sigmoid(
            dinv_b * (w + up_b) + b2_ref[...])


def _gcn(A1, A2, E1, E2, W1, W2, b1r, b2r):
    out11 = pl.BlockSpec((N, 1), lambda p, k: (0, 0))
    return pl.pallas_call(
        _gcn_body,
        grid=(3, GK),
        in_specs=[
            pl.BlockSpec((N, B), lambda p, k: (0, jnp.where(p == 0, k, GK - 1))),
            pl.BlockSpec((N, B),
                         lambda p, k: (0, jnp.where(p == 0, 0,
                                                    jnp.where(p == 1, k, GK - 1)))),
            pl.BlockSpec((B, N), lambda p, k: (jnp.where(p == 2, k, 0), 0)),
            pl.BlockSpec((B, D_IN), lambda p, k: (jnp.where(p == 0, k, GK - 1), 0)),
            pl.BlockSpec((B, D_IN),
                         lambda p, k: (jnp.where(p == 0, 0,
                                                 jnp.where(p == 1, k, GK - 1)), 0)),
            pl.BlockSpec((D_IN, D_HID), lambda p, k: (0, 0)),
            pl.BlockSpec((D_HID, 1), lambda p, k: (0, 0)),
            pl.BlockSpec((1, D_HID), lambda p, k: (0, 0)),
            pl.BlockSpec((1, 1), lambda p, k: (0, 0)),
        ],
        out_specs=[out11, out11, out11, out11, out11],
        out_shape=[jax.ShapeDtypeStruct((N, 1), jnp.float32)] * 5,
        scratch_shapes=[
            pltpu.VMEM((N, D_HID), jnp.float32),
            pltpu.VMEM((N, D_HID), jnp.float32),
            pltpu.VMEM((N, 1), jnp.bfloat16),
        ],
    )(A1, A2, A2, E1, E2, W1, W2, b1r, b2r)


def _tail_body(state_ref, a1blk_ref, upxn_ref, g2_ref, upx_ref, dinvx_ref,
               wh_ref, wf_ref, wp_ref, biash_ref, b2_ref, out_ref):
    ix = state_ref[0]
    iy = state_ref[1]
    gids = (jax.lax.broadcasted_iota(jnp.int32, (32, 128), 0) * 128
            + jax.lax.broadcasted_iota(jnp.int32, (32, 128), 1))
    blk = a1blk_ref[...]
    rsel = jax.lax.broadcasted_iota(jnp.int32, (8, N), 0) == (ix % 8)
    row = jnp.sum(jnp.where(rsel, blk, 0.0), axis=0, keepdims=True)
    dot = jnp.dot(row, upxn_ref[...], preferred_element_type=jnp.float32)[0, 0]
    upx = upx_ref[...]
    upxi = jnp.sum(jnp.where(gids == ix, upx, 0.0))
    dxi = jnp.sum(jnp.where(gids == ix, dinvx_ref[...], 0.0))
    b2 = b2_ref[0, 0]
    gx = jax.nn.sigmoid(dxi * (dot + upxi) + b2)

    g2 = g2_ref[...]
    gy = jnp.sum(jnp.where(gids == iy, g2, 0.0))

    h = jax.nn.sigmoid(wh_ref[0, 0] * gx + wh_ref[0, 1] * gy + biash_ref[0, 0])
    wf = wf_ref[0, 0]
    f = jnp.exp(gx * wf * gy)

    num = g2 * gy
    den = jnp.maximum(jnp.sqrt(g2 * g2), 1e-8) * jnp.maximum(
        jnp.sqrt(gy * gy), 1e-8)
    sims = num / den
    work = sims
    f_oppo = jnp.float32(0.0)
    for _ in range(K_OPP):
        mval = jnp.max(work)
        first = jnp.min(jnp.where(work == mval, gids, N))
        sel = gids == first
        opp = jnp.sum(jnp.where(sel, g2, 0.0))
        f_oppo = f_oppo + jnp.exp(gx * wf * opp)
        work = jnp.where(sel, -jnp.inf, work)

    i_ratio = f / f_oppo
    wp = wp_ref[0, 0]
    z1 = wp * h
    z2 = wp * i_ratio
    mz = jnp.maximum(z1, z2)
    e1 = jnp.exp(z1 - mz)
    e2 = jnp.exp(z2 - mz)
    s = e1 + e2
    out_ref[...] = jnp.concatenate(
        [(e1 / s).reshape(1, 1), (e2 / s).reshape(1, 1)], axis=1)


def _tail(state32, A1, upxn, g2, upx32, dinvx32, W_h, W_f, W_p, biash_r, b2r):
    grid_spec = pltpu.PrefetchScalarGridSpec(
        num_scalar_prefetch=1,
        grid=(1,),
        in_specs=[
            pl.BlockSpec((8, N), lambda i, st: (st[0] // 8, 0)),
            pl.BlockSpec((N, 1), lambda i, st: (0, 0)),
            pl.BlockSpec((32, 128), lambda i, st: (0, 0)),
            pl.BlockSpec((32, 128), lambda i, st: (0, 0)),
            pl.BlockSpec((32, 128), lambda i, st: (0, 0)),
            pl.BlockSpec((1, 2), lambda i, st: (0, 0)),
            pl.BlockSpec((1, 1), lambda i, st: (0, 0)),
            pl.BlockSpec((1, 1), lambda i, st: (0, 0)),
            pl.BlockSpec((1, 1), lambda i, st: (0, 0)),
            pl.BlockSpec((1, 1), lambda i, st: (0, 0)),
        ],
        out_specs=pl.BlockSpec((1, 2), lambda i, st: (0, 0)),
    )
    return pl.pallas_call(
        _tail_body,
        grid_spec=grid_spec,
        out_shape=jax.ShapeDtypeStruct((1, 2), jnp.float32),
    )(state32, A1, upxn, g2, upx32, dinvx32, W_h, W_f, W_p, biash_r, b2r)


def kernel(first_embeddings, second_embeddings, state, A1, A2, W1, b1, W2, b2,
           W_h, W_f, W_p, bias_h):
    state32 = state.astype(jnp.int32)
    b1r = b1.reshape(1, D_HID)
    b2r = b2.reshape(1, 1)
    biash_r = bias_h.reshape(1, 1)
    up_x, dinv_x, up_y, dinv_y, G_y = _gcn(
        A1, A2, first_embeddings, second_embeddings, W1, W2, b1r, b2r)
    return _tail(state32, A1, up_x, G_y.reshape(32, 128),
                 up_x.reshape(32, 128), dinv_x.reshape(32, 128),
                 W_h, W_f, W_p, biash_r, b2r)


# speedup vs baseline: 1.0999x; 1.0999x over previous
import jax
import jax.numpy as jnp
from jax.experimental import pallas as pl
from jax.experimental.pallas import tpu as pltpu

N = 4096
D_IN = 256
D_HID = 128
K_OPP = 11
B1 = 512
GK1 = N // B1
B2 = 512
GK2 = N // B2


def _layer1_stripe(k, a_ref, e_ref, W1_ref, W2_ref, b1_ref,
                   up_ref, dinv_ref, z_s, mp_s):
    a = a_ref[...]
    ab = a.astype(jnp.bfloat16)
    colr = jnp.sum(a, axis=0, keepdims=True)
    dinv_c = jnp.transpose(1.0 / (colr + 1.0))
    dinv_ref[pl.ds(k * B1, B1), :] = dinv_c
    m = jnp.dot(e_ref[...].astype(jnp.bfloat16), W1_ref[...].astype(jnp.bfloat16),
                preferred_element_type=jnp.float32)
    mp = dinv_c * m
    mp_s[pl.ds(k * B1, B1), :] = mp
    zp = jnp.dot(ab, mp.astype(jnp.bfloat16), preferred_element_type=jnp.float32)

    @pl.when(k == 0)
    def _():
        z_s[...] = zp

    @pl.when(k != 0)
    def _():
        z_s[...] += zp

    @pl.when(k == GK1 - 1)
    def _():
        dinv = dinv_ref[...]
        xm = jax.nn.sigmoid(dinv * (z_s[...] + mp_s[...]) + b1_ref[...])
        u = jnp.dot(xm, W2_ref[...], preferred_element_type=jnp.float32)
        up_ref[...] = dinv * u


def _pass1_body(A1_ref, A2_ref, E1_ref, E2_ref, W1_ref, W2_ref, b1_ref,
                upx_ref, dinvx_ref, upy_ref, dinvy_ref, z_s, mp_s):
    p = pl.program_id(0)
    k = pl.program_id(1)

    @pl.when(p == 0)
    def _():
        _layer1_stripe(k, A1_ref, E1_ref, W1_ref, W2_ref, b1_ref,
                       upx_ref, dinvx_ref, z_s, mp_s)

    @pl.when(p == 1)
    def _():
        _layer1_stripe(k, A2_ref, E2_ref, W1_ref, W2_ref, b1_ref,
                       upy_ref, dinvy_ref, z_s, mp_s)


def _pass1(A1, A2, E1, E2, W1, W2, b1r):
    out11 = pl.BlockSpec((N, 1), lambda p, k: (0, 0))
    return pl.pallas_call(
        _pass1_body,
        grid=(2, GK1),
        in_specs=[
            pl.BlockSpec((N, B1), lambda p, k: (0, jnp.where(p == 0, k, GK1 - 1))),
            pl.BlockSpec((N, B1), lambda p, k: (0, jnp.where(p == 1, k, 0))),
            pl.BlockSpec((B1, D_IN), lambda p, k: (jnp.where(p == 0, k, GK1 - 1), 0)),
            pl.BlockSpec((B1, D_IN), lambda p, k: (jnp.where(p == 1, k, 0), 0)),
            pl.BlockSpec((D_IN, D_HID), lambda p, k: (0, 0)),
            pl.BlockSpec((D_HID, 1), lambda p, k: (0, 0)),
            pl.BlockSpec((1, D_HID), lambda p, k: (0, 0)),
        ],
        out_specs=[out11, out11, out11, out11],
        out_shape=[jax.ShapeDtypeStruct((N, 1), jnp.float32)] * 4,
        scratch_shapes=[
            pltpu.VMEM((N, D_HID), jnp.float32),
            pltpu.VMEM((N, D_HID), jnp.float32),
        ],
    )(A1, A2, E1, E2, W1, W2, b1r)


def _pass2_body(A_ref, upf_ref, ups_ref, dinv_ref, b2_ref, G_ref):
    ab = A_ref[...].astype(jnp.bfloat16)
    w = jnp.dot(ab, upf_ref[...].astype(jnp.bfloat16),
                preferred_element_type=jnp.float32)
    G_ref[...] = jax.nn.sigmoid(
        dinv_ref[...] * (w + ups_ref[...]) + b2_ref[...])


def _pass2(A, up, dinv, b2r):
    return pl.pallas_call(
        _pass2_body,
        grid=(GK2,),
        in_specs=[
            pl.BlockSpec((B2, N), lambda k: (k, 0)),
            pl.BlockSpec((N, 1), lambda k: (0, 0)),
            pl.BlockSpec((B2, 1), lambda k: (k, 0)),
            pl.BlockSpec((B2, 1), lambda k: (k, 0)),
            pl.BlockSpec((1, 1), lambda k: (0, 0)),
        ],
        out_specs=pl.BlockSpec((B2, 1), lambda k: (k, 0)),
        out_shape=jax.ShapeDtypeStruct((N, 1), jnp.float32),
    )(A, up, up, dinv, b2r)


def _tail_body(state_ref, a1blk_ref, upxn_ref, g2_ref, upx_ref, dinvx_ref,
               wh_ref, wf_ref, wp_ref, biash_ref, b2_ref, out_ref):
    ix = state_ref[0]
    iy = state_ref[1]
    gids = (jax.lax.broadcasted_iota(jnp.int32, (32, 128), 0) * 128
            + jax.lax.broadcasted_iota(jnp.int32, (32, 128), 1))
    blk = a1blk_ref[...]
    rsel = jax.lax.broadcasted_iota(jnp.int32, (8, N), 0) == (ix % 8)
    row = jnp.sum(jnp.where(rsel, blk, 0.0), axis=0, keepdims=True)
    dot = jnp.dot(row, upxn_ref[...], preferred_element_type=jnp.float32)[0, 0]
    upx = upx_ref[...]
    upxi = jnp.sum(jnp.where(gids == ix, upx, 0.0))
    dxi = jnp.sum(jnp.where(gids == ix, dinvx_ref[...], 0.0))
    b2 = b2_ref[0, 0]
    gx = jax.nn.sigmoid(dxi * (dot + upxi) + b2)

    g2 = g2_ref[...]
    gy = jnp.sum(jnp.where(gids == iy, g2, 0.0))

    h = jax.nn.sigmoid(wh_ref[0, 0] * gx + wh_ref[0, 1] * gy + biash_ref[0, 0])
    wf = wf_ref[0, 0]
    f = jnp.exp(gx * wf * gy)

    num = g2 * gy
    den = jnp.maximum(jnp.sqrt(g2 * g2), 1e-8) * jnp.maximum(
        jnp.sqrt(gy * gy), 1e-8)
    sims = num / den
    work = sims
    f_oppo = jnp.float32(0.0)
    for _ in range(K_OPP):
        mval = jnp.max(work)
        first = jnp.min(jnp.where(work == mval, gids, N))
        sel = gids == first
        opp = jnp.sum(jnp.where(sel, g2, 0.0))
        f_oppo = f_oppo + jnp.exp(gx * wf * opp)
        work = jnp.where(sel, -jnp.inf, work)

    i_ratio = f / f_oppo
    wp = wp_ref[0, 0]
    z1 = wp * h
    z2 = wp * i_ratio
    mz = jnp.maximum(z1, z2)
    e1 = jnp.exp(z1 - mz)
    e2 = jnp.exp(z2 - mz)
    s = e1 + e2
    out_ref[...] = jnp.concatenate(
        [(e1 / s).reshape(1, 1), (e2 / s).reshape(1, 1)], axis=1)


def _tail(state32, A1, upxn, g2, upx32, dinvx32, W_h, W_f, W_p, biash_r, b2r):
    grid_spec = pltpu.PrefetchScalarGridSpec(
        num_scalar_prefetch=1,
        grid=(1,),
        in_specs=[
            pl.BlockSpec((8, N), lambda i, st: (st[0] // 8, 0)),
            pl.BlockSpec((N, 1), lambda i, st: (0, 0)),
            pl.BlockSpec((32, 128), lambda i, st: (0, 0)),
            pl.BlockSpec((32, 128), lambda i, st: (0, 0)),
            pl.BlockSpec((32, 128), lambda i, st: (0, 0)),
            pl.BlockSpec((1, 2), lambda i, st: (0, 0)),
            pl.BlockSpec((1, 1), lambda i, st: (0, 0)),
            pl.BlockSpec((1, 1), lambda i, st: (0, 0)),
            pl.BlockSpec((1, 1), lambda i, st: (0, 0)),
            pl.BlockSpec((1, 1), lambda i, st: (0, 0)),
        ],
        out_specs=pl.BlockSpec((1, 2), lambda i, st: (0, 0)),
    )
    return pl.pallas_call(
        _tail_body,
        grid_spec=grid_spec,
        out_shape=jax.ShapeDtypeStruct((1, 2), jnp.float32),
    )(state32, A1, upxn, g2, upx32, dinvx32, W_h, W_f, W_p, biash_r, b2r)


def kernel(first_embeddings, second_embeddings, state, A1, A2, W1, b1, W2, b2,
           W_h, W_f, W_p, bias_h):
    state32 = state.astype(jnp.int32)
    b1r = b1.reshape(1, D_HID)
    b2r = b2.reshape(1, 1)
    biash_r = bias_h.reshape(1, 1)
    up_x, dinv_x, up_y, dinv_y = _pass1(
        A1, A2, first_embeddings, second_embeddings, W1, W2, b1r)
    G_y = _pass2(A2, up_y, dinv_y, b2r)
    return _tail(state32, A1, up_x, G_y.reshape(32, 128),
                 up_x.reshape(32, 128), dinv_x.reshape(32, 128),
                 W_h, W_f, W_p, biash_r, b2r)


# speedup vs baseline: 1.1063x; 1.0058x over previous
import jax
import jax.numpy as jnp
from jax.experimental import pallas as pl
from jax.experimental.pallas import tpu as pltpu

N = 4096
D_IN = 256
D_HID = 128
K_OPP = 11
B1 = 1024
GK1 = N // B1
B2 = 512
GK2 = N // B2


def _pass1_body(A_ref, E_ref, W1_ref, W2_ref, b1_ref, up_ref, dinv_ref, z_s, mp_s):
    k = pl.program_id(0)
    a = A_ref[...]
    ab = a.astype(jnp.bfloat16)
    colr = jnp.sum(a, axis=0, keepdims=True)
    dinv_c = jnp.transpose(1.0 / (colr + 1.0))
    dinv_ref[pl.ds(k * B1, B1), :] = dinv_c
    m = jnp.dot(E_ref[...].astype(jnp.bfloat16), W1_ref[...].astype(jnp.bfloat16),
                preferred_element_type=jnp.float32)
    mp = dinv_c * m
    mp_s[pl.ds(k * B1, B1), :] = mp
    zp = jnp.dot(ab, mp.astype(jnp.bfloat16), preferred_element_type=jnp.float32)

    @pl.when(k == 0)
    def _():
        z_s[...] = zp

    @pl.when(k != 0)
    def _():
        z_s[...] += zp

    @pl.when(k == GK1 - 1)
    def _():
        dinv = dinv_ref[...]
        xm = jax.nn.sigmoid(dinv * (z_s[...] + mp_s[...]) + b1_ref[...])
        u = jnp.dot(xm, W2_ref[...], preferred_element_type=jnp.float32)
        up_ref[...] = dinv * u


def _pass1(A, E, W1, W2, b1r):
    return pl.pallas_call(
        _pass1_body,
        grid=(GK1,),
        in_specs=[
            pl.BlockSpec((N, B1), lambda k: (0, k)),
            pl.BlockSpec((B1, D_IN), lambda k: (k, 0)),
            pl.BlockSpec((D_IN, D_HID), lambda k: (0, 0)),
            pl.BlockSpec((D_HID, 1), lambda k: (0, 0)),
            pl.BlockSpec((1, D_HID), lambda k: (0, 0)),
        ],
        out_specs=[
            pl.BlockSpec((N, 1), lambda k: (0, 0)),
            pl.BlockSpec((N, 1), lambda k: (0, 0)),
        ],
        out_shape=[
            jax.ShapeDtypeStruct((N, 1), jnp.float32),
            jax.ShapeDtypeStruct((N, 1), jnp.float32),
        ],
        scratch_shapes=[
            pltpu.VMEM((N, D_HID), jnp.float32),
            pltpu.VMEM((N, D_HID), jnp.float32),
        ],
    )(A, E, W1, W2, b1r)


def _pass2_body(A_ref, upf_ref, ups_ref, dinv_ref, b2_ref, G_ref):
    ab = A_ref[...].astype(jnp.bfloat16)
    w = jnp.dot(ab, upf_ref[...].astype(jnp.bfloat16),
                preferred_element_type=jnp.float32)
    G_ref[...] = jax.nn.sigmoid(
        dinv_ref[...] * (w + ups_ref[...]) + b2_ref[...])


def _pass2(A, up, dinv, b2r):
    return pl.pallas_call(
        _pass2_body,
        grid=(GK2,),
        in_specs=[
            pl.BlockSpec((B2, N), lambda k: (k, 0)),
            pl.BlockSpec((N, 1), lambda k: (0, 0)),
            pl.BlockSpec((B2, 1), lambda k: (k, 0)),
            pl.BlockSpec((B2, 1), lambda k: (k, 0)),
            pl.BlockSpec((1, 1), lambda k: (0, 0)),
        ],
        out_specs=pl.BlockSpec((B2, 1), lambda k: (k, 0)),
        out_shape=jax.ShapeDtypeStruct((N, 1), jnp.float32),
    )(A, up, up, dinv, b2r)


def _tail_body(state_ref, a1blk_ref, upxn_ref, g2_ref, upx_ref, dinvx_ref,
               wh_ref, wf_ref, wp_ref, biash_ref, b2_ref, out_ref):
    ix = state_ref[0]
    iy = state_ref[1]
    gids = (jax.lax.broadcasted_iota(jnp.int32, (32, 128), 0) * 128
            + jax.lax.broadcasted_iota(jnp.int32, (32, 128), 1))
    blk = a1blk_ref[...]
    rsel = jax.lax.broadcasted_iota(jnp.int32, (8, N), 0) == (ix % 8)
    row = jnp.sum(jnp.where(rsel, blk, 0.0), axis=0, keepdims=True)
    dot = jnp.dot(row, upxn_ref[...], preferred_element_type=jnp.float32)[0, 0]
    upx = upx_ref[...]
    upxi = jnp.sum(jnp.where(gids == ix, upx, 0.0))
    dxi = jnp.sum(jnp.where(gids == ix, dinvx_ref[...], 0.0))
    b2 = b2_ref[0, 0]
    gx = jax.nn.sigmoid(dxi * (dot + upxi) + b2)

    g2 = g2_ref[...]
    gy = jnp.sum(jnp.where(gids == iy, g2, 0.0))

    h = jax.nn.sigmoid(wh_ref[0, 0] * gx + wh_ref[0, 1] * gy + biash_ref[0, 0])
    wf = wf_ref[0, 0]
    f = jnp.exp(gx * wf * gy)

    num = g2 * gy
    den = jnp.maximum(jnp.sqrt(g2 * g2), 1e-8) * jnp.maximum(
        jnp.sqrt(gy * gy), 1e-8)
    sims = num / den
    work = sims
    f_oppo = jnp.float32(0.0)
    for _ in range(K_OPP):
        mval = jnp.max(work)
        first = jnp.min(jnp.where(work == mval, gids, N))
        sel = gids == first
        opp = jnp.sum(jnp.where(sel, g2, 0.0))
        f_oppo = f_oppo + jnp.exp(gx * wf * opp)
        work = jnp.where(sel, -jnp.inf, work)

    i_ratio = f / f_oppo
    wp = wp_ref[0, 0]
    z1 = wp * h
    z2 = wp * i_ratio
    mz = jnp.maximum(z1, z2)
    e1 = jnp.exp(z1 - mz)
    e2 = jnp.exp(z2 - mz)
    s = e1 + e2
    out_ref[...] = jnp.concatenate(
        [(e1 / s).reshape(1, 1), (e2 / s).reshape(1, 1)], axis=1)


def _tail(state32, A1, upxn, g2, upx32, dinvx32, W_h, W_f, W_p, biash_r, b2r):
    grid_spec = pltpu.PrefetchScalarGridSpec(
        num_scalar_prefetch=1,
        grid=(1,),
        in_specs=[
            pl.BlockSpec((8, N), lambda i, st: (st[0] // 8, 0)),
            pl.BlockSpec((N, 1), lambda i, st: (0, 0)),
            pl.BlockSpec((32, 128), lambda i, st: (0, 0)),
            pl.BlockSpec((32, 128), lambda i, st: (0, 0)),
            pl.BlockSpec((32, 128), lambda i, st: (0, 0)),
            pl.BlockSpec((1, 2), lambda i, st: (0, 0)),
            pl.BlockSpec((1, 1), lambda i, st: (0, 0)),
            pl.BlockSpec((1, 1), lambda i, st: (0, 0)),
            pl.BlockSpec((1, 1), lambda i, st: (0, 0)),
            pl.BlockSpec((1, 1), lambda i, st: (0, 0)),
        ],
        out_specs=pl.BlockSpec((1, 2), lambda i, st: (0, 0)),
    )
    return pl.pallas_call(
        _tail_body,
        grid_spec=grid_spec,
        out_shape=jax.ShapeDtypeStruct((1, 2), jnp.float32),
    )(state32, A1, upxn, g2, upx32, dinvx32, W_h, W_f, W_p, biash_r, b2r)


def kernel(first_embeddings, second_embeddings, state, A1, A2, W1, b1, W2, b2,
           W_h, W_f, W_p, bias_h):
    state32 = state.astype(jnp.int32)
    b1r = b1.reshape(1, D_HID)
    b2r = b2.reshape(1, 1)
    biash_r = bias_h.reshape(1, 1)
    up_x, dinv_x = _pass1(A1, first_embeddings, W1, W2, b1r)
    up_y, dinv_y = _pass1(A2, second_embeddings, W1, W2, b1r)
    G_y = _pass2(A2, up_y, dinv_y, b2r)
    return _tail(state32, A1, up_x, G_y.reshape(32, 128),
                 up_x.reshape(32, 128), dinv_x.reshape(32, 128),
                 W_h, W_f, W_p, biash_r, b2r)
